# baseline (speedup 1.0000x reference)
import functools

import jax
import jax.numpy as jnp
from jax import lax
from jax.experimental import pallas as pl
from jax.experimental.pallas import tpu as pltpu
from jax.experimental.pallas import tpu_sc as plsc

B, N, D, K = 2, 10000, 128, 16
NC, NS = 2, 16
NW = NC * NS
C = 8
NPB = 10240
NP = B * NPB
NCHUNK = NP // C
CPW = NCHUNK // NW
SB = 8
NSB = CPW // SB
SBR = SB * C
RPW = CPW * C
LANES = 16
DB = D // LANES
REM = N - (NS - 1) * RPW
FULLSB = REM // SBR
PREM = REM - FULLSB * SBR

_mesh = plsc.VectorSubcoreMesh(core_axis_name="c", subcore_axis_name="s")

_BCAST_DNUMS = lax.GatherDimensionNumbers(
    offset_dims=(), collapsed_slice_dims=(0,), start_index_map=(0,))


def _bcast_lane(v, k):
    idx = jnp.full((LANES, 1), k, jnp.int32)
    return lax.gather(v, idx, _BCAST_DNUMS, (1,),
                      mode=lax.GatherScatterMode.PROMISE_IN_BOUNDS)


def _pipeline(table_hbm, gidx_hbm, w_hbm, idx_v, w_v, rows_v, outsb_v,
              sem_c, sem_g, drain_sb, store_sb):
    cid = lax.axis_index("c")
    sid = lax.axis_index("s")
    chunk0 = (cid * NS + sid) * CPW

    def _stage_i(b, buf):
        return pltpu.make_async_copy(
            gidx_hbm.at[pl.ds((chunk0 + b * SB), SB)], idx_v.at[buf], sem_c)

    def _stage_w(b, buf):
        return pltpu.make_async_copy(
            w_hbm.at[pl.ds((chunk0 + b * SB), SB)], w_v.at[buf], sem_c)

    def _stage_start(b, buf):
        _stage_i(b, buf).start()
        _stage_w(b, buf).start()

    def _stage_wait():
        _stage_i(0, 0).wait()
        _stage_w(0, 0).wait()

    def _gather(buf_c, c, buf_g):
        idx = idx_v.at[buf_c, c]
        return pltpu.make_async_copy(table_hbm.at[idx], rows_v.at[buf_g],
                                     sem_g)

    _stage_start(0, 0)
    _stage_wait()
    _gather(0, 0, 0).start()

    def sb_body(b, carry):
        pb = lax.rem(b, 2)
        drain_sb(b)

        @pl.when(b + 1 < NSB)
        def _():
            _stage_start(b + 1, 1 - pb)

        def chunk_body(c, carry2):
            g = b * SB + c
            gb = lax.rem(g, 2)

            @pl.when(c < SB - 1)
            def _():
                _gather(pb, c + 1, 1 - gb).start()

            @pl.when((c == SB - 1) & (b + 1 < NSB))
            def _():
                _stage_wait()
                _gather(1 - pb, 0, 1 - gb).start()

            _gather(0, 0, gb).wait()

            for r in range(C):
                srow = w_v[pb, c, pl.ds(r * K, K)]
                accs = [None] * DB
                for k in range(K):
                    w = _bcast_lane(srow, k)
                    for db in range(DB):
                        xv = rows_v[gb, r * K + k, pl.ds(db * LANES, LANES)]
                        if accs[db] is None:
                            accs[db] = w * xv
                        else:
                            accs[db] = accs[db] + w * xv
                for db in range(DB):
                    outsb_v[pb, c * C + r, pl.ds(db * LANES, LANES)] = accs[db]
            return carry2

        lax.fori_loop(0, SB, chunk_body, 0)
        store_sb(b, pb)
        return carry

    lax.fori_loop(0, NSB, sb_body, 0)


_SCRATCH = [
    pltpu.VMEM((2, SB, C * K), jnp.int32),
    pltpu.VMEM((2, SB, C * K), jnp.float32),
    pltpu.VMEM((2, C * K, D), jnp.float32),
    pltpu.VMEM((2, SBR, D), jnp.float32),
    pltpu.SemaphoreType.DMA,
    pltpu.SemaphoreType.DMA,
    pltpu.SemaphoreType.DMA,
]


@functools.partial(
    pl.kernel,
    out_type=jax.ShapeDtypeStruct((NP, D), jnp.float32),
    mesh=_mesh,
    scratch_types=_SCRATCH,
)
def _hop1(table_hbm, gidx_hbm, w_hbm, out_hbm, idx_v, w_v, rows_v, outsb_v,
          sem_c, sem_g, sem_o):
    cid = lax.axis_index("c")
    sid = lax.axis_index("s")
    row0 = (cid * NS + sid) * RPW

    def _store(b, buf):
        return pltpu.make_async_copy(
            outsb_v.at[buf], out_hbm.at[pl.ds(row0 + b * SBR, SBR)], sem_o)

    def drain_sb(b):
        @pl.when(b >= 2)
        def _():
            _store(0, 0).wait()

    def store_sb(b, buf):
        _store(b, buf).start()

    _pipeline(table_hbm, gidx_hbm, w_hbm, idx_v, w_v, rows_v, outsb_v,
              sem_c, sem_g, drain_sb, store_sb)
    _store(0, 0).wait()
    _store(0, 0).wait()


@functools.partial(
    pl.kernel,
    out_type=jax.ShapeDtypeStruct((B * 3 * N, D), jnp.float32),
    mesh=_mesh,
    scratch_types=_SCRATCH + [
        pltpu.SemaphoreType.DMA,
        pltpu.SemaphoreType.DMA,
    ],
)
def _hop2(xp_hbm, x1_hbm, gidx_hbm, w_hbm, out_hbm, idx_v, w_v, rows_v,
          outsb_v, sem_c, sem_g, sem_o, sem_t, sem_p):
    cid = lax.axis_index("c")
    sid = lax.axis_index("s")
    wid = cid * NS + sid
    row0 = wid * RPW
    n0 = sid * RPW
    tail = sid == NS - 1
    dst2 = (cid * 3 + 2) * N + n0

    def _store(b, buf):
        return pltpu.make_async_copy(
            outsb_v.at[buf], out_hbm.at[pl.ds(dst2 + b * SBR, SBR)], sem_o)

    def drain_sb(b):
        @pl.when((b >= 2) & (jnp.logical_not(tail) | (b <= FULLSB + 1)))
        def _():
            _store(0, 0).wait()

        @pl.when(tail & (b == FULLSB + 2))
        def _():
            pltpu.make_async_copy(
                outsb_v.at[0, pl.ds(0, PREM)],
                out_hbm.at[pl.ds(dst2 + FULLSB * SBR, PREM)], sem_t).wait()

    def store_sb(b, buf):
        @pl.when(jnp.logical_not(tail) | (b < FULLSB))
        def _():
            _store(b, buf).start()

        @pl.when(tail & (b == FULLSB))
        def _():
            pltpu.make_async_copy(
                outsb_v.at[buf, pl.ds(0, PREM)],
                out_hbm.at[pl.ds(dst2 + FULLSB * SBR, PREM)], sem_t).start()

    def _plane_copies(nrows):
        cpx = pltpu.make_async_copy(
            xp_hbm.at[pl.ds(row0, nrows)],
            out_hbm.at[pl.ds((cid * 3 + 0) * N + n0, nrows)], sem_p)
        cpx1 = pltpu.make_async_copy(
            x1_hbm.at[pl.ds(row0, nrows)],
            out_hbm.at[pl.ds((cid * 3 + 1) * N + n0, nrows)], sem_p)
        return cpx, cpx1

    @pl.when(jnp.logical_not(tail))
    def _():
        a, b = _plane_copies(RPW)
        a.start()
        b.start()

    @pl.when(tail)
    def _():
        a, b = _plane_copies(REM)
        a.start()
        b.start()

    _pipeline(x1_hbm, gidx_hbm, w_hbm, idx_v, w_v, rows_v, outsb_v,
              sem_c, sem_g, drain_sb, store_sb)

    @pl.when(jnp.logical_not(tail))
    def _():
        _store(0, 0).wait()
        _store(0, 0).wait()
        a, b = _plane_copies(RPW)
        a.wait()
        b.wait()

    @pl.when(tail)
    def _():
        a, b = _plane_copies(REM)
        a.wait()
        b.wait()


def kernel(x, s1, t1):
    pad = NPB - N
    padidx = ((jnp.arange(pad * K, dtype=jnp.int32) * 61) % N).reshape(
        1, pad, K)
    offs = (jnp.arange(B, dtype=jnp.int32) * NPB)[:, None, None]
    gidx = jnp.concatenate(
        [t1.astype(jnp.int32), jnp.broadcast_to(padidx, (B, pad, K))],
        axis=1) + offs
    gidx = gidx.reshape(NCHUNK, C * K)
    sf = jnp.pad(s1, ((0, 0), (0, pad), (0, 0))).reshape(NCHUNK, C * K)
    xp = jnp.pad(x, ((0, 0), (0, pad), (0, 0))).reshape(NP, D)
    x1 = _hop1(xp, gidx, sf)
    h = _hop2(xp, x1, gidx, sf)
    return h.reshape(B, 3, N, D)


# speedup vs baseline: 3.0718x; 3.0718x over previous
import functools

import jax
import jax.numpy as jnp
from jax import lax
from jax.experimental import pallas as pl
from jax.experimental.pallas import tpu as pltpu
from jax.experimental.pallas import tpu_sc as plsc

B, N, D, K = 2, 10000, 128, 16
NC, NS = 2, 16
NW = NC * NS
C = 8
NCHUNK = 2560
CPW = NCHUNK // NW
SB = 8
NSB = CPW // SB
NP = NCHUNK * C
LANES = 16
DB = D // LANES

_mesh = plsc.VectorSubcoreMesh(core_axis_name="c", subcore_axis_name="s")

_BCAST_DNUMS = lax.GatherDimensionNumbers(
    offset_dims=(), collapsed_slice_dims=(0,), start_index_map=(0,))


def _bcast_lane(v, k):
    idx = jnp.full((LANES, 1), k, jnp.int32)
    return lax.gather(v, idx, _BCAST_DNUMS, (1,),
                      mode=lax.GatherScatterMode.PROMISE_IN_BOUNDS)


@functools.partial(
    pl.kernel,
    out_type=jax.ShapeDtypeStruct((NP, D), jnp.float32),
    mesh=_mesh,
    scratch_types=[
        pltpu.VMEM((2, SB, C * K), jnp.int32),
        pltpu.VMEM((2, SB, C * K), jnp.float32),
        pltpu.VMEM((2, C * K, D), jnp.float32),
        pltpu.VMEM((2, SB * C, D), jnp.float32),
        pltpu.SemaphoreType.DMA,
        pltpu.SemaphoreType.DMA,
        pltpu.SemaphoreType.DMA,
    ],
)
def _hop(table_hbm, gidx_hbm, w_hbm, out_hbm, idx_v, w_v, rows_v, outsb_v,
         sem_c, sem_g, sem_o):
    cid = lax.axis_index("c")
    sid = lax.axis_index("s")
    wid = cid * NS + sid
    chunk0 = wid * CPW

    def _stage_i(b, buf):
        return pltpu.make_async_copy(
            gidx_hbm.at[pl.ds((chunk0 + b * SB), SB)], idx_v.at[buf], sem_c)

    def _stage_w(b, buf):
        return pltpu.make_async_copy(
            w_hbm.at[pl.ds((chunk0 + b * SB), SB)], w_v.at[buf], sem_c)

    def _stage_start(b, buf):
        _stage_i(b, buf).start()
        _stage_w(b, buf).start()

    def _stage_wait():
        _stage_i(0, 0).wait()
        _stage_w(0, 0).wait()

    def _gather(buf_c, c, buf_g):
        idx = idx_v.at[buf_c, c]
        return pltpu.make_async_copy(table_hbm.at[idx], rows_v.at[buf_g],
                                     sem_g)

    def _store(b, buf):
        return pltpu.make_async_copy(
            outsb_v.at[buf], out_hbm.at[pl.ds((chunk0 + b * SB) * C, SB * C)],
            sem_o)

    _stage_start(0, 0)
    _stage_wait()
    _gather(0, 0, 0).start()

    def sb_body(b, carry):
        pb = lax.rem(b, 2)

        @pl.when(b >= 2)
        def _():
            _store(0, 0).wait()

        @pl.when(b + 1 < NSB)
        def _():
            _stage_start(b + 1, 1 - pb)

        def chunk_body(c, carry2):
            g = b * SB + c
            gb = lax.rem(g, 2)

            @pl.when(c < SB - 1)
            def _():
                _gather(pb, c + 1, 1 - gb).start()

            @pl.when((c == SB - 1) & (b + 1 < NSB))
            def _():
                _stage_wait()
                _gather(1 - pb, 0, 1 - gb).start()

            _gather(0, 0, gb).wait()

            for r in range(C):
                srow = w_v[pb, c, pl.ds(r * K, K)]
                accs = [None] * DB
                for k in range(K):
                    w = _bcast_lane(srow, k)
                    for db in range(DB):
                        xv = rows_v[gb, r * K + k, pl.ds(db * LANES, LANES)]
                        if accs[db] is None:
                            accs[db] = w * xv
                        else:
                            accs[db] = accs[db] + w * xv
                for db in range(DB):
                    outsb_v[pb, c * C + r, pl.ds(db * LANES, LANES)] = accs[db]
            return carry2

        lax.fori_loop(0, SB, chunk_body, 0)
        _store(b, pb).start()
        return carry

    lax.fori_loop(0, NSB, sb_body, 0)
    _store(0, 0).wait()
    _store(0, 0).wait()


def kernel(x, s1, t1):
    pad = NP - B * N
    padidx = (jnp.arange(pad * K, dtype=jnp.int32) * 61) % (B * N)
    xf = x.reshape(B * N, D)
    offs = (jnp.arange(B, dtype=jnp.int32) * N)[:, None, None]
    gidx = (t1.astype(jnp.int32) + offs).reshape(B * N * K)
    gidx = jnp.concatenate([gidx, padidx]).reshape(NCHUNK, C * K)
    sf = jnp.pad(s1.reshape(B * N * K), (0, pad * K)).reshape(NCHUNK, C * K)
    x1 = _hop(xf, gidx, sf)
    x2 = _hop(x1, gidx, sf)
    h = jnp.stack([xf, x1[: B * N], x2[: B * N]], axis=0)
    return h.reshape(3, B, N, D).transpose(1, 0, 2, 3)


# speedup vs baseline: 3.0727x; 1.0003x over previous
import functools

import jax
import jax.numpy as jnp
from jax import lax
from jax.experimental import pallas as pl
from jax.experimental.pallas import tpu as pltpu
from jax.experimental.pallas import tpu_sc as plsc

B, N, D, K = 2, 10000, 128, 16
NC, NS = 2, 16
NW = NC * NS
C = 8
NCHUNK = 2560
CPW = NCHUNK // NW
SB = 8
NSB = CPW // SB
NP = NCHUNK * C
LANES = 16
DB = D // LANES

_mesh = plsc.VectorSubcoreMesh(core_axis_name="c", subcore_axis_name="s")

_BCAST_DNUMS = lax.GatherDimensionNumbers(
    offset_dims=(), collapsed_slice_dims=(0,), start_index_map=(0,))


def _bcast_lane(v, k):
    idx = jnp.full((LANES, 1), k, jnp.int32)
    return lax.gather(v, idx, _BCAST_DNUMS, (1,),
                      mode=lax.GatherScatterMode.PROMISE_IN_BOUNDS)


@functools.partial(
    pl.kernel,
    out_type=jax.ShapeDtypeStruct((NP, D), jnp.float32),
    mesh=_mesh,
    scratch_types=[
        pltpu.VMEM((2, SB, C * K), jnp.int32),
        pltpu.VMEM((2, SB, C * K), jnp.float32),
        pltpu.VMEM((2, C * K, D), jnp.float32),
        pltpu.VMEM((2, SB * C, D), jnp.float32),
        pltpu.SemaphoreType.DMA,
        pltpu.SemaphoreType.DMA,
        pltpu.SemaphoreType.DMA,
    ],
)
def _hop(table_hbm, gidx_hbm, w_hbm, out_hbm, idx_v, w_v, rows_v, outsb_v,
         sem_c, sem_g, sem_o):
    cid = lax.axis_index("c")
    sid = lax.axis_index("s")
    wid = cid * NS + sid
    chunk0 = wid * CPW

    def _stage_i(b, buf):
        return pltpu.make_async_copy(
            gidx_hbm.at[pl.ds((chunk0 + b * SB), SB)], idx_v.at[buf], sem_c)

    def _stage_w(b, buf):
        return pltpu.make_async_copy(
            w_hbm.at[pl.ds((chunk0 + b * SB), SB)], w_v.at[buf], sem_c)

    def _stage_start(b, buf):
        _stage_i(b, buf).start()
        _stage_w(b, buf).start()

    def _stage_wait():
        _stage_i(0, 0).wait()
        _stage_w(0, 0).wait()

    def _gather(buf_c, c, buf_g):
        idx = idx_v.at[buf_c, c]
        return pltpu.make_async_copy(table_hbm.at[idx], rows_v.at[buf_g],
                                     sem_g)

    def _store(b, buf):
        return pltpu.make_async_copy(
            outsb_v.at[buf], out_hbm.at[pl.ds((chunk0 + b * SB) * C, SB * C)],
            sem_o)

    _stage_start(0, 0)
    _stage_wait()
    _gather(0, 0, 0).start()

    def sb_body(b, carry):
        pb = lax.rem(b, 2)

        @pl.when(b >= 2)
        def _():
            _store(0, 0).wait()

        @pl.when(b + 1 < NSB)
        def _():
            _stage_start(b + 1, 1 - pb)

        def chunk_body(c, carry2):
            g = b * SB + c
            gb = lax.rem(g, 2)

            @pl.when(c < SB - 1)
            def _():
                _gather(pb, c + 1, 1 - gb).start()

            @pl.when((c == SB - 1) & (b + 1 < NSB))
            def _():
                _stage_wait()
                _gather(1 - pb, 0, 1 - gb).start()

            _gather(0, 0, gb).wait()

            for r in range(C):
                srow = w_v[pb, c, pl.ds(r * K, K)]
                accs = [None] * DB
                for k in range(K):
                    w = _bcast_lane(srow, k)
                    for db in range(DB):
                        xv = rows_v[gb, r * K + k, pl.ds(db * LANES, LANES)]
                        if accs[db] is None:
                            accs[db] = w * xv
                        else:
                            accs[db] = accs[db] + w * xv
                for db in range(DB):
                    outsb_v[pb, c * C + r, pl.ds(db * LANES, LANES)] = accs[db]
            return carry2

        lax.fori_loop(0, SB, chunk_body, 0)
        _store(b, pb).start()
        return carry

    lax.fori_loop(0, NSB, sb_body, 0)
    _store(0, 0).wait()
    _store(0, 0).wait()


def kernel(x, s1, t1):
    pad = NP - B * N
    padidx = (jnp.arange(pad * K, dtype=jnp.int32) * 61) % (B * N)
    xf = x.reshape(B * N, D)
    offs = (jnp.arange(B, dtype=jnp.int32) * N)[:, None, None]
    gidx = (t1.astype(jnp.int32) + offs).reshape(B * N * K)
    gidx = jnp.concatenate([gidx, padidx]).reshape(NCHUNK, C * K)
    sf = jnp.pad(s1.reshape(B * N * K), (0, pad * K)).reshape(NCHUNK, C * K)
    x1 = _hop(xf, gidx, sf)
    x2 = _hop(x1, gidx, sf)
    return jnp.concatenate(
        [x.reshape(B, 1, N, D),
         x1[: B * N].reshape(B, 1, N, D),
         x2[: B * N].reshape(B, 1, N, D)], axis=1)


# speedup vs baseline: 3.1334x; 1.0197x over previous
import functools

import jax
import jax.numpy as jnp
from jax import lax
from jax.experimental import pallas as pl
from jax.experimental.pallas import tpu as pltpu
from jax.experimental.pallas import tpu_sc as plsc

B, N, D, K = 2, 10000, 128, 16
NC, NS = 2, 16
NW = NC * NS
C = 8
NCHUNK = 2560
CPW = NCHUNK // NW
SB = 8
NSB = CPW // SB
NP = NCHUNK * C
LANES = 16
DB = D // LANES

_mesh = plsc.VectorSubcoreMesh(core_axis_name="c", subcore_axis_name="s")

_BCAST_DNUMS = lax.GatherDimensionNumbers(
    offset_dims=(), collapsed_slice_dims=(0,), start_index_map=(0,))


def _bcast_lane(v, k):
    idx = jnp.full((LANES, 1), k, jnp.int32)
    return lax.gather(v, idx, _BCAST_DNUMS, (1,),
                      mode=lax.GatherScatterMode.PROMISE_IN_BOUNDS)


@functools.partial(
    pl.kernel,
    out_type=jax.ShapeDtypeStruct((NP, D), jnp.float32),
    mesh=_mesh,
    scratch_types=[
        pltpu.VMEM((2, SB, C * K), jnp.int32),
        pltpu.VMEM((2, SB, C * K), jnp.float32),
        pltpu.VMEM((2, C * K, D), jnp.float32),
        pltpu.VMEM((2, SB * C, D), jnp.float32),
        pltpu.SemaphoreType.DMA,
        pltpu.SemaphoreType.DMA,
        pltpu.SemaphoreType.DMA,
    ],
)
def _hop(table_hbm, gidx_hbm, w_hbm, out_hbm, idx_v, w_v, rows_v, outsb_v,
         sem_c, sem_g, sem_o):
    cid = lax.axis_index("c")
    sid = lax.axis_index("s")
    wid = cid * NS + sid
    chunk0 = wid * CPW

    def _stage_i(b, buf):
        return pltpu.make_async_copy(
            gidx_hbm.at[pl.ds((chunk0 + b * SB), SB)], idx_v.at[buf], sem_c)

    def _stage_w(b, buf):
        return pltpu.make_async_copy(
            w_hbm.at[pl.ds((chunk0 + b * SB), SB)], w_v.at[buf], sem_c)

    def _stage_start(b, buf):
        _stage_i(b, buf).start()
        _stage_w(b, buf).start()

    def _stage_wait():
        _stage_i(0, 0).wait()
        _stage_w(0, 0).wait()

    def _gather(buf_c, c, buf_g):
        idx = idx_v.at[buf_c, c]
        return pltpu.make_async_copy(table_hbm.at[idx], rows_v.at[buf_g],
                                     sem_g)

    def _store(b, buf):
        return pltpu.make_async_copy(
            outsb_v.at[buf], out_hbm.at[pl.ds((chunk0 + b * SB) * C, SB * C)],
            sem_o)

    _stage_start(0, 0)
    _stage_wait()
    _gather(0, 0, 0).start()

    def sb_body(b, carry):
        pb = lax.rem(b, 2)

        @pl.when(b >= 2)
        def _():
            _store(0, 0).wait()

        @pl.when(b + 1 < NSB)
        def _():
            _stage_start(b + 1, 1 - pb)

        def chunk_body(c, carry2):
            g = b * SB + c
            gb = lax.rem(g, 2)

            @pl.when(c < SB - 1)
            def _():
                _gather(pb, c + 1, 1 - gb).start()

            @pl.when((c == SB - 1) & (b + 1 < NSB))
            def _():
                _stage_wait()
                _gather(1 - pb, 0, 1 - gb).start()

            _gather(0, 0, gb).wait()

            for r in range(C):
                srow = w_v[pb, c, pl.ds(r * K, K)]
                accs = [None] * DB
                for k in range(K):
                    w = _bcast_lane(srow, k)
                    for db in range(DB):
                        xv = rows_v[gb, r * K + k, pl.ds(db * LANES, LANES)]
                        if accs[db] is None:
                            accs[db] = w * xv
                        else:
                            accs[db] = accs[db] + w * xv
                for db in range(DB):
                    outsb_v[pb, c * C + r, pl.ds(db * LANES, LANES)] = accs[db]
            return carry2

        lax.fori_loop(0, SB, chunk_body, 0)
        _store(b, pb).start()
        return carry

    lax.fori_loop(0, NSB, sb_body, 0)
    _store(0, 0).wait()
    _store(0, 0).wait()


def kernel(x, s1, t1):
    pad = NP - B * N
    padidx = (jnp.arange(pad * K, dtype=jnp.int32) * 61) % (B * N)
    xf = x.reshape(B * N, D)
    offs = (jnp.arange(B, dtype=jnp.int32) * N)[:, None, None]
    gidx = (t1.astype(jnp.int32) + offs).reshape(B * N * K)
    gidx = jnp.concatenate([gidx, padidx]).reshape(NCHUNK, C * K)
    sf = jnp.pad(s1.reshape(B * N * K), (0, pad * K)).reshape(NCHUNK, C * K)
    x1 = _hop(xf, gidx, sf)
    out = jnp.zeros((B, 3, N, D), jnp.float32)
    out = out.at[:, 0].set(x)
    out = out.at[:, 1].set(x1[: B * N].reshape(B, N, D))
    x2 = _hop(x1, gidx, sf)
    return out.at[:, 2].set(x2[: B * N].reshape(B, N, D))


# speedup vs baseline: 3.1355x; 1.0007x over previous
import functools

import jax
import jax.numpy as jnp
from jax import lax
from jax.experimental import pallas as pl
from jax.experimental.pallas import tpu as pltpu
from jax.experimental.pallas import tpu_sc as plsc

B, N, D, K = 2, 10000, 128, 16
NC, NS = 2, 16
NW = NC * NS
C = 8
NCHUNK = 2560
CPW = NCHUNK // NW
SB = 8
NSB = CPW // SB
NP = NCHUNK * C
LANES = 16
DB = D // LANES

_mesh = plsc.VectorSubcoreMesh(core_axis_name="c", subcore_axis_name="s")

_BCAST_DNUMS = lax.GatherDimensionNumbers(
    offset_dims=(), collapsed_slice_dims=(0,), start_index_map=(0,))


def _bcast_lane(v, k):
    idx = jnp.full((LANES, 1), k, jnp.int32)
    return lax.gather(v, idx, _BCAST_DNUMS, (1,),
                      mode=lax.GatherScatterMode.PROMISE_IN_BOUNDS)


@functools.partial(
    pl.kernel,
    out_type=jax.ShapeDtypeStruct((NP, D), jnp.float32),
    mesh=_mesh,
    scratch_types=[
        pltpu.VMEM((2, SB, C * K), jnp.int32),
        pltpu.VMEM((2, SB, C * K), jnp.float32),
        pltpu.VMEM((2, C * K, D), jnp.float32),
        pltpu.VMEM((2, SB * C, D), jnp.float32),
        pltpu.SemaphoreType.DMA,
        pltpu.SemaphoreType.DMA,
        pltpu.SemaphoreType.DMA,
    ],
)
def _hop(table_hbm, gidx_hbm, w_hbm, out_hbm, idx_v, w_v, rows_v, outsb_v,
         sem_c, sem_g, sem_o):
    cid = lax.axis_index("c")
    sid = lax.axis_index("s")
    wid = cid * NS + sid
    chunk0 = wid * CPW

    def _stage_i(b, buf):
        return pltpu.make_async_copy(
            gidx_hbm.at[pl.ds((chunk0 + b * SB), SB)], idx_v.at[buf], sem_c)

    def _stage_w(b, buf):
        return pltpu.make_async_copy(
            w_hbm.at[pl.ds((chunk0 + b * SB), SB)], w_v.at[buf], sem_c)

    def _stage_start(b, buf):
        _stage_i(b, buf).start()
        _stage_w(b, buf).start()

    def _stage_wait():
        _stage_i(0, 0).wait()
        _stage_w(0, 0).wait()

    def _gather(buf_c, c, buf_g):
        idx = idx_v.at[buf_c, c]
        return pltpu.make_async_copy(table_hbm.at[idx], rows_v.at[buf_g],
                                     sem_g)

    def _store(b, buf):
        return pltpu.make_async_copy(
            outsb_v.at[buf], out_hbm.at[pl.ds((chunk0 + b * SB) * C, SB * C)],
            sem_o)

    _stage_start(0, 0)
    _stage_wait()
    _gather(0, 0, 0).start()

    def sb_body(b, carry):
        pb = lax.rem(b, 2)

        @pl.when(b >= 2)
        def _():
            _store(0, 0).wait()

        @pl.when(b + 1 < NSB)
        def _():
            _stage_start(b + 1, 1 - pb)

        def chunk_body(c, carry2):
            g = b * SB + c
            gb = lax.rem(g, 2)

            @pl.when(c < SB - 1)
            def _():
                _gather(pb, c + 1, 1 - gb).start()

            @pl.when((c == SB - 1) & (b + 1 < NSB))
            def _():
                _stage_wait()
                _gather(1 - pb, 0, 1 - gb).start()

            _gather(0, 0, gb).wait()

            for r in range(C):
                srow = w_v[pb, c, pl.ds(r * K, K)]
                accs = [None] * DB
                for k in range(K):
                    w = _bcast_lane(srow, k)
                    for db in range(DB):
                        xv = rows_v[gb, r * K + k, pl.ds(db * LANES, LANES)]
                        if accs[db] is None:
                            accs[db] = w * xv
                        else:
                            accs[db] = accs[db] + w * xv
                for db in range(DB):
                    outsb_v[pb, c * C + r, pl.ds(db * LANES, LANES)] = accs[db]
            return carry2

        lax.fori_loop(0, SB, chunk_body, 0)
        _store(b, pb).start()
        return carry

    lax.fori_loop(0, NSB, sb_body, 0)
    _store(0, 0).wait()
    _store(0, 0).wait()


def kernel(x, s1, t1):
    pad = NP - B * N
    padidx = (jnp.arange(pad * K, dtype=jnp.int32) * 61) % (B * N)
    xf = x.reshape(B * N, D)
    offs = (jnp.arange(B, dtype=jnp.int32) * N)[:, None, None]
    gidx = (t1.astype(jnp.int32) + offs).reshape(B * N * K)
    gidx = jnp.concatenate([gidx, padidx]).reshape(NCHUNK, C * K)
    sf = jnp.pad(s1.reshape(B * N * K), (0, pad * K)).reshape(NCHUNK, C * K)
    out = jnp.zeros((B, 3, N, D), jnp.float32)
    out = out.at[:, 0].set(x)
    x1 = _hop(xf, gidx, sf)
    out = out.at[:, 1].set(x1[: B * N].reshape(B, N, D))
    x2 = _hop(x1, gidx, sf)
    return out.at[:, 2].set(x2[: B * N].reshape(B, N, D))
